# baseline (speedup 1.0000x reference)
import jax
import jax.numpy as jnp
from jax import lax
from jax.experimental import pallas as pl
from jax.experimental.pallas import tpu as pltpu
from jax.experimental.pallas import tpu_sc as plsc

IN_F = 6
OUT_F = 11
LANES = 16
IMG_SIZE = 128


def _make_body(nc, rpw, n_rows):
    half = rpw // 2

    def body(x_hbm, out_hbm, in_v, out_v, sem_in, sem_out):
        wid = lax.axis_index("s") * nc + lax.axis_index("c")
        row0 = wid * rpw

        in_copies = [
            pltpu.async_copy(
                x_hbm.at[pl.ds(f * n_rows + row0, rpw)],
                in_v.at[pl.ds(f * rpw, rpw)],
                sem_in,
            )
            for f in range(IN_F)
        ]

        inv = jnp.float32(1.0 / IMG_SIZE)
        zero = jnp.zeros((LANES,), jnp.float32)

        def make_chunk(o):
            g = [in_v[pl.ds(f * rpw + o, LANES)] for f in range(IN_F)]
            prob = g[4]
            cid = g[5].astype(jnp.int32)
            cid = jnp.minimum(jnp.maximum(cid, 0), 8)
            c = (cid >= 3).astype(jnp.int32) + (cid >= 6).astype(jnp.int32)
            s = cid - 3 * c
            outs = (
                g[0] * inv, g[1] * inv, g[2] * inv, g[3] * inv,
                jnp.where(c == 0, prob, zero),
                jnp.where(c == 1, prob, zero),
                jnp.where(c == 2, prob, zero),
                jnp.where(s == 0, prob, zero),
                jnp.where(s == 1, prob, zero),
                jnp.where(s == 2, prob, zero),
                prob,
            )
            for f in range(OUT_F):
                out_v[pl.ds(f * rpw + o, LANES)] = outs[f]

        for c in in_copies:
            c.wait()

        @plsc.parallel_loop(0, half, LANES, unroll=8)
        def _first(o):
            make_chunk(o)

        out_a = [
            pltpu.async_copy(
                out_v.at[pl.ds(f * rpw, half)],
                out_hbm.at[pl.ds(f * n_rows + row0, half)],
                sem_out,
            )
            for f in range(OUT_F)
        ]

        @plsc.parallel_loop(half, rpw, LANES, unroll=8)
        def _second(o):
            make_chunk(o)

        out_b = [
            pltpu.async_copy(
                out_v.at[pl.ds(f * rpw + half, half)],
                out_hbm.at[pl.ds(f * n_rows + row0 + half, half)],
                sem_out,
            )
            for f in range(OUT_F)
        ]
        for c in out_a + out_b:
            c.wait()

    return body


def kernel(x):
    n, obj_num, feat = x.shape
    rows = n * obj_num
    mesh = plsc.VectorSubcoreMesh(core_axis_name="c", subcore_axis_name="s")
    nw = mesh.num_cores * mesh.num_subcores
    rpw = rows // nw

    k = pl.kernel(
        _make_body(mesh.num_cores, rpw, rows),
        out_type=jax.ShapeDtypeStruct((rows * OUT_F,), jnp.float32),
        mesh=mesh,
        compiler_params=pltpu.CompilerParams(needs_layout_passes=False),
        scratch_types=[
            pltpu.VMEM((rpw * IN_F,), jnp.float32),
            pltpu.VMEM((rpw * OUT_F,), jnp.float32),
            pltpu.SemaphoreType.DMA,
            pltpu.SemaphoreType.DMA,
        ],
    )
    xt = jnp.transpose(x, (2, 1, 0)).reshape(-1)
    out_flat = k(xt)
    return jnp.transpose(out_flat.reshape(OUT_F, obj_num, n), (2, 1, 0))


# speedup vs baseline: 1.0214x; 1.0214x over previous
import jax
import jax.numpy as jnp
from jax import lax
from jax.experimental import pallas as pl
from jax.experimental.pallas import tpu as pltpu
from jax.experimental.pallas import tpu_sc as plsc

IN_F = 6
OUT_F = 11
LANES = 16
IMG_SIZE = 128


def _make_body(nc, rpw, n_rows):
    half = rpw // 2

    def body(x_hbm, out_hbm, in_v, out_v, sem_in, sem_out):
        wid = lax.axis_index("s") * nc + lax.axis_index("c")
        row0 = wid * rpw

        in_copies = [
            pltpu.async_copy(
                x_hbm.at[pl.ds(f * n_rows + row0, rpw)],
                in_v.at[pl.ds(f * rpw, rpw)],
                sem_in,
            )
            for f in range(IN_F)
        ]

        inv = jnp.float32(1.0 / IMG_SIZE)
        zero = jnp.zeros((LANES,), jnp.float32)

        def make_chunk(o):
            g = [in_v[pl.ds(f * rpw + o, LANES)] for f in range(IN_F)]
            prob = g[4]
            cid = g[5].astype(jnp.int32)
            cid = jnp.minimum(jnp.maximum(cid, 0), 8)
            c = (cid >= 3).astype(jnp.int32) + (cid >= 6).astype(jnp.int32)
            s = cid - 3 * c
            outs = (
                g[0] * inv, g[1] * inv, g[2] * inv, g[3] * inv,
                jnp.where(c == 0, prob, zero),
                jnp.where(c == 1, prob, zero),
                jnp.where(c == 2, prob, zero),
                jnp.where(s == 0, prob, zero),
                jnp.where(s == 1, prob, zero),
                jnp.where(s == 2, prob, zero),
                prob,
            )
            for f in range(OUT_F):
                out_v[pl.ds(f * rpw + o, LANES)] = outs[f]

        for c in in_copies:
            c.wait()

        @plsc.parallel_loop(0, half, LANES, unroll=4)
        def _first(o):
            make_chunk(o)

        out_a = [
            pltpu.async_copy(
                out_v.at[pl.ds(f * rpw, half)],
                out_hbm.at[pl.ds(f * n_rows + row0, half)],
                sem_out,
            )
            for f in range(OUT_F)
        ]

        @plsc.parallel_loop(half, rpw, LANES, unroll=4)
        def _second(o):
            make_chunk(o)

        out_b = [
            pltpu.async_copy(
                out_v.at[pl.ds(f * rpw + half, half)],
                out_hbm.at[pl.ds(f * n_rows + row0 + half, half)],
                sem_out,
            )
            for f in range(OUT_F)
        ]
        for c in out_a + out_b:
            c.wait()

    return body


def kernel(x):
    n, obj_num, feat = x.shape
    rows = n * obj_num
    mesh = plsc.VectorSubcoreMesh(core_axis_name="c", subcore_axis_name="s")
    nw = mesh.num_cores * mesh.num_subcores
    rpw = rows // nw

    k = pl.kernel(
        _make_body(mesh.num_cores, rpw, rows),
        out_type=jax.ShapeDtypeStruct((rows * OUT_F,), jnp.float32),
        mesh=mesh,
        compiler_params=pltpu.CompilerParams(needs_layout_passes=False),
        scratch_types=[
            pltpu.VMEM((rpw * IN_F,), jnp.float32),
            pltpu.VMEM((rpw * OUT_F,), jnp.float32),
            pltpu.SemaphoreType.DMA,
            pltpu.SemaphoreType.DMA,
        ],
    )
    xt = jnp.transpose(x, (2, 1, 0)).reshape(-1)
    out_flat = k(xt)
    return jnp.transpose(out_flat.reshape(OUT_F, obj_num, n), (2, 1, 0))
